# baseline (speedup 1.0000x reference)
import jax
import jax.numpy as jnp
from jax.experimental import pallas as pl
from jax.experimental.pallas import tpu as pltpu

BETA = 0.1
VBLK = 65536


def _fused_kernel(ids_ref, s_ref, o_ref, boost_ref, lists_ref, counts_ref):
    j = pl.program_id(0)
    nblk = pl.num_programs(0)
    n_ent = ids_ref.shape[0]

    @pl.when(j == 0)
    def _():
        def zero_counts(b, _):
            counts_ref[b] = 0
            return 0

        jax.lax.fori_loop(0, nblk, zero_counts, 0)

        def bin_one(i, _):
            e = ids_ref[i]
            blk = jax.lax.shift_right_logical(e, 16)
            c = counts_ref[blk]
            lists_ref[blk, c] = e
            counts_ref[blk] = c + 1
            return 0

        jax.lax.fori_loop(0, n_ent, bin_one, 0)

    boost_ref[...] = jnp.zeros_like(boost_ref)
    lane_iota = jax.lax.broadcasted_iota(jnp.int32, (8, 128), 1)
    blk_lo = j * VBLK

    def scatter_one(i, _):
        e = lists_ref[j, i] - blk_lo
        base = pl.multiple_of((e // 128) * 128, 128)
        tile = boost_ref[:, pl.ds(base, 128)]
        tile = jnp.where(lane_iota == e - base, jnp.asarray(BETA, tile.dtype),
                         tile)
        boost_ref[:, pl.ds(base, 128)] = tile
        return 0

    jax.lax.fori_loop(0, counts_ref[j], scatter_one, 0)

    b = boost_ref[...]
    for k in range(s_ref.shape[0] // 8):
        o_ref[8 * k:8 * (k + 1), :] = s_ref[8 * k:8 * (k + 1), :] + b


def kernel(input_ids, scores, cur_len, entity_token_ids):
    del input_ids, cur_len
    batch, vocab = scores.shape
    nblk = pl.cdiv(vocab, VBLK)
    n_ent = entity_token_ids.shape[0]

    return pl.pallas_call(
        _fused_kernel,
        out_shape=jax.ShapeDtypeStruct((batch, vocab), scores.dtype),
        grid=(nblk,),
        in_specs=[
            pl.BlockSpec(memory_space=pltpu.SMEM),
            pl.BlockSpec((batch, VBLK), lambda j: (0, j)),
        ],
        out_specs=pl.BlockSpec((batch, VBLK), lambda j: (0, j)),
        scratch_shapes=[
            pltpu.VMEM((8, VBLK), scores.dtype),
            pltpu.SMEM((nblk, n_ent), jnp.int32),
            pltpu.SMEM((nblk,), jnp.int32),
        ],
        compiler_params=pltpu.CompilerParams(
            dimension_semantics=("arbitrary",),
        ),
    )(entity_token_ids.astype(jnp.int32), scores)


# speedup vs baseline: 1.0429x; 1.0429x over previous
import functools

import jax
import jax.numpy as jnp
from jax.experimental import pallas as pl
from jax.experimental.pallas import tpu as pltpu

BETA = 0.1
VBLK = 65536


def _fused_kernel(vocab, ids_ref, s_hbm, o_hbm, in_buf, out_buf, in_edge,
                  out_edge, boost_ref, in_sems, out_sems, edge_sems):
    nfull = vocab // VBLK
    rem = vocab - nfull * VBLK

    def in_copy(j):
        return pltpu.make_async_copy(
            s_hbm.at[:, pl.ds(j * VBLK, VBLK)], in_buf.at[j % 2],
            in_sems.at[j % 2])

    def out_copy(j):
        return pltpu.make_async_copy(
            out_buf.at[j % 2], o_hbm.at[:, pl.ds(j * VBLK, VBLK)],
            out_sems.at[j % 2])

    def edge_in_copy():
        return pltpu.make_async_copy(
            s_hbm.at[:, pl.ds(nfull * VBLK, rem)], in_edge, edge_sems.at[0])

    def edge_out_copy():
        return pltpu.make_async_copy(
            out_edge, o_hbm.at[:, pl.ds(nfull * VBLK, rem)], edge_sems.at[1])

    in_copy(0).start()
    if nfull > 1:
        in_copy(1).start()
    if rem:
        edge_in_copy().start()

    boost_ref[...] = jnp.zeros_like(boost_ref)
    lane_iota = jax.lax.broadcasted_iota(jnp.int32, (1, 128), 1)

    def scatter_one(i, _):
        e = ids_ref[i]
        base = pl.multiple_of((e // 128) * 128, 128)
        row = boost_ref[0:1, pl.ds(base, 128)]
        row = jnp.where(lane_iota == e - base, jnp.asarray(BETA, row.dtype),
                        row)
        boost_ref[0:1, pl.ds(base, 128)] = row
        return 0

    jax.lax.fori_loop(0, ids_ref.shape[0], scatter_one, 0)

    for j in range(nfull):
        if j >= 2:
            out_copy(j - 2).wait()
        in_copy(j).wait()
        out_buf[j % 2] = (in_buf[j % 2]
                          + boost_ref[0:1, pl.ds(j * VBLK, VBLK)])
        out_copy(j).start()
        if j + 2 < nfull:
            in_copy(j + 2).start()

    if rem:
        edge_in_copy().wait()
        out_edge[...] = (in_edge[...]
                         + boost_ref[0:1, pl.ds(nfull * VBLK, rem)])
        edge_out_copy().start()

    if nfull >= 2:
        out_copy(nfull - 2).wait()
    if nfull >= 1:
        out_copy(nfull - 1).wait()
    if rem:
        edge_out_copy().wait()


def kernel(input_ids, scores, cur_len, entity_token_ids):
    del input_ids, cur_len
    batch, vocab = scores.shape
    rem = vocab - (vocab // VBLK) * VBLK

    return pl.pallas_call(
        functools.partial(_fused_kernel, vocab),
        out_shape=jax.ShapeDtypeStruct((batch, vocab), scores.dtype),
        in_specs=[
            pl.BlockSpec(memory_space=pltpu.SMEM),
            pl.BlockSpec(memory_space=pltpu.MemorySpace.HBM),
        ],
        out_specs=pl.BlockSpec(memory_space=pltpu.MemorySpace.HBM),
        scratch_shapes=[
            pltpu.VMEM((2, batch, VBLK), scores.dtype),
            pltpu.VMEM((2, batch, VBLK), scores.dtype),
            pltpu.VMEM((batch, max(rem, 1)), scores.dtype),
            pltpu.VMEM((batch, max(rem, 1)), scores.dtype),
            pltpu.VMEM((1, pl.cdiv(vocab, 128) * 128), scores.dtype),
            pltpu.SemaphoreType.DMA((2,)),
            pltpu.SemaphoreType.DMA((2,)),
            pltpu.SemaphoreType.DMA((2,)),
        ],
    )(entity_token_ids.astype(jnp.int32), scores)


# speedup vs baseline: 1.0872x; 1.0425x over previous
import functools

import jax
import jax.numpy as jnp
from jax.experimental import pallas as pl
from jax.experimental.pallas import tpu as pltpu

BETA = 0.1
VBLK = 65536


def _fused_kernel(vocab, ids_ref, s_hbm, o_hbm, in_buf, out_buf, in_edge,
                  out_edge, boost_ref, in_sems, out_sems, edge_sems):
    nfull = vocab // VBLK
    rem = vocab - nfull * VBLK

    def in_copy(j):
        return pltpu.make_async_copy(
            s_hbm.at[:, pl.ds(j * VBLK, VBLK)], in_buf.at[j % 3],
            in_sems.at[j % 3])

    def out_copy(j):
        return pltpu.make_async_copy(
            out_buf.at[j % 3], o_hbm.at[:, pl.ds(j * VBLK, VBLK)],
            out_sems.at[j % 3])

    def edge_in_copy():
        return pltpu.make_async_copy(
            s_hbm.at[:, pl.ds(nfull * VBLK, rem)], in_edge, edge_sems.at[0])

    def edge_out_copy():
        return pltpu.make_async_copy(
            out_edge, o_hbm.at[:, pl.ds(nfull * VBLK, rem)], edge_sems.at[1])

    for j in range(min(3, nfull)):
        in_copy(j).start()
    if rem:
        edge_in_copy().start()

    boost_ref[...] = jnp.zeros_like(boost_ref)
    lane_iota = jax.lax.broadcasted_iota(jnp.int32, (1, 128), 1)

    def scatter_one(i, _):
        e = ids_ref[i]
        base = pl.multiple_of((e // 128) * 128, 128)
        row = boost_ref[0:1, pl.ds(base, 128)]
        row = jnp.where(lane_iota == e - base, jnp.asarray(BETA, row.dtype),
                        row)
        boost_ref[0:1, pl.ds(base, 128)] = row
        return 0

    jax.lax.fori_loop(0, ids_ref.shape[0], scatter_one, 0)

    for j in range(nfull):
        if j >= 3:
            out_copy(j - 3).wait()
        in_copy(j).wait()
        out_buf[j % 3] = (in_buf[j % 3]
                          + boost_ref[0:1, pl.ds(j * VBLK, VBLK)])
        out_copy(j).start()
        if j + 3 < nfull:
            in_copy(j + 3).start()

    if rem:
        edge_in_copy().wait()
        out_edge[...] = (in_edge[...]
                         + boost_ref[0:1, pl.ds(nfull * VBLK, rem)])
        edge_out_copy().start()

    for j in range(max(0, nfull - 3), nfull):
        out_copy(j).wait()
    if rem:
        edge_out_copy().wait()


def kernel(input_ids, scores, cur_len, entity_token_ids):
    del input_ids, cur_len
    batch, vocab = scores.shape
    rem = vocab - (vocab // VBLK) * VBLK

    return pl.pallas_call(
        functools.partial(_fused_kernel, vocab),
        out_shape=jax.ShapeDtypeStruct((batch, vocab), scores.dtype),
        in_specs=[
            pl.BlockSpec(memory_space=pltpu.SMEM),
            pl.BlockSpec(memory_space=pltpu.MemorySpace.HBM),
        ],
        out_specs=pl.BlockSpec(memory_space=pltpu.MemorySpace.HBM),
        scratch_shapes=[
            pltpu.VMEM((3, batch, VBLK), scores.dtype),
            pltpu.VMEM((3, batch, VBLK), scores.dtype),
            pltpu.VMEM((batch, max(rem, 1)), scores.dtype),
            pltpu.VMEM((batch, max(rem, 1)), scores.dtype),
            pltpu.VMEM((1, pl.cdiv(vocab, 128) * 128), scores.dtype),
            pltpu.SemaphoreType.DMA((3,)),
            pltpu.SemaphoreType.DMA((3,)),
            pltpu.SemaphoreType.DMA((2,)),
        ],
    )(entity_token_ids.astype(jnp.int32), scores)
